# Initial kernel scaffold; baseline (speedup 1.0000x reference)
import jax
import jax.numpy as jnp
from jax.experimental import pallas as pl


def kernel(inputs, affine_weight, affine_bias, head_w, head_b):
    raise NotImplementedError("write your pallas kernel here")


# speedup vs baseline: 1.6340x; 1.6340x over previous
import functools

import jax
import jax.numpy as jnp
from jax.experimental import pallas as pl
from jax.experimental.pallas import tpu as pltpu

EPS = 1e-5


def _round_up(x, m):
    return ((x + m - 1) // m) * m


def _fused_kernel(x_ref, w_ref, b_ref, invw_ref, hwT_ref, hb_ref, o_ref):
    x = x_ref[0].astype(jnp.float32)
    inv_s = 1.0 / x.shape[0]

    mean = jnp.sum(x, axis=0, keepdims=True) * inv_s
    diff = x - mean
    var = jnp.sum(diff * diff, axis=0, keepdims=True) * inv_s
    inv_std = jax.lax.rsqrt(var + EPS)
    std = (var + EPS) * inv_std

    xn = diff * inv_std * w_ref[...] + b_ref[...]

    pred = jnp.dot(hwT_ref[...], xn, preferred_element_type=jnp.float32)
    pred = pred + hb_ref[...]

    scale = invw_ref[...] * std
    res = (pred - b_ref[...]) * scale + mean
    o_ref[0] = res[: o_ref.shape[1], :].astype(o_ref.dtype)


@functools.partial(jax.jit, static_argnames=("patch_len", "stride", "pred_len"))
def _rose_forward(inputs, affine_weight, affine_bias, head_w, head_b,
                  *, patch_len, stride, pred_len):
    B, S, K = inputs.shape
    assert S >= patch_len, "seq_len < patch_len not supported"

    num_patch = (max(S, patch_len) - patch_len) // stride + 1
    tgt_len = patch_len + stride * (num_patch - 1)
    s_begin = S - tgt_len

    N_pad = _round_up(pred_len, 128)
    Kb = 256
    K_pad = _round_up(K, Kb)
    KB = K_pad // Kb
    out_dtype = inputs.dtype

    hw = head_w.astype(jnp.float32)
    hw_pad = jnp.pad(hw, ((0, 0), (0, N_pad - pred_len)))
    t_idx = (s_begin
             + jnp.arange(num_patch)[:, None] * stride
             + jnp.arange(patch_len)[None, :]).reshape(-1)
    hwT = jnp.zeros((S, N_pad), jnp.float32).at[t_idx].add(hw_pad).T
    hb_col = jnp.pad(head_b.astype(jnp.float32),
                     (0, N_pad - pred_len)).reshape(N_pad, 1)

    w_l = jnp.pad(affine_weight.astype(jnp.float32), (0, K_pad - K),
                  constant_values=1.0).reshape(1, K_pad)
    b_l = jnp.pad(affine_bias.astype(jnp.float32),
                  (0, K_pad - K)).reshape(1, K_pad)
    invw_l = 1.0 / (w_l + EPS * EPS)

    out = pl.pallas_call(
        _fused_kernel,
        out_shape=jax.ShapeDtypeStruct((B, pred_len, K), out_dtype),
        grid=(B, KB),
        in_specs=[
            pl.BlockSpec((1, S, Kb), lambda i, j: (i, 0, j)),
            pl.BlockSpec((1, Kb), lambda i, j: (0, j)),
            pl.BlockSpec((1, Kb), lambda i, j: (0, j)),
            pl.BlockSpec((1, Kb), lambda i, j: (0, j)),
            pl.BlockSpec((N_pad, S), lambda i, j: (0, 0)),
            pl.BlockSpec((N_pad, 1), lambda i, j: (0, 0)),
        ],
        out_specs=pl.BlockSpec((1, pred_len, Kb), lambda i, j: (i, 0, j)),
        compiler_params=pltpu.CompilerParams(
            dimension_semantics=("parallel", "parallel")),
    )(inputs, w_l, b_l, invw_l, hwT, hb_col)

    xe = jnp.zeros((), jnp.float32)
    xq = jnp.zeros((), jnp.float32)
    return out, xe, xq


def kernel(inputs, affine_weight, affine_bias, head_w, head_b):
    return _rose_forward(inputs, affine_weight, affine_bias, head_w, head_b,
                         patch_len=16, stride=8, pred_len=96)


# speedup vs baseline: 1.8519x; 1.1334x over previous
import functools

import jax
import jax.numpy as jnp
from jax.experimental import pallas as pl
from jax.experimental.pallas import tpu as pltpu

EPS = 1e-5


def _round_up(x, m):
    return ((x + m - 1) // m) * m


def _fused_kernel(x_ref, w_ref, b_ref, invw_ref, hwT_ref, hb_ref, o_ref):
    x = x_ref[0].astype(jnp.float32)
    inv_s = 1.0 / x.shape[0]

    mean = jnp.sum(x, axis=0, keepdims=True) * inv_s
    diff = x - mean
    var = jnp.sum(diff * diff, axis=0, keepdims=True) * inv_s
    inv_std = jax.lax.rsqrt(var + EPS)
    std = (var + EPS) * inv_std

    xn = diff * inv_std * w_ref[...] + b_ref[...]

    pred = jnp.dot(hwT_ref[...], xn, preferred_element_type=jnp.float32)
    pred = pred + hb_ref[...]

    scale = invw_ref[...] * std
    res = (pred - b_ref[...]) * scale + mean
    o_ref[0] = res[: o_ref.shape[1], :].astype(o_ref.dtype)


@functools.partial(jax.jit, static_argnames=("patch_len", "stride", "pred_len"))
def _rose_forward(inputs, affine_weight, affine_bias, head_w, head_b,
                  *, patch_len, stride, pred_len):
    B, S, K = inputs.shape
    assert S >= patch_len, "seq_len < patch_len not supported"

    num_patch = (max(S, patch_len) - patch_len) // stride + 1
    tgt_len = patch_len + stride * (num_patch - 1)
    s_begin = S - tgt_len

    N_pad = _round_up(pred_len, 128)
    Kb = 256
    K_pad = _round_up(K, Kb)
    KB = K_pad // Kb
    out_dtype = inputs.dtype

    hw = head_w.astype(jnp.float32)
    hw_pad = jnp.pad(hw, ((0, 0), (0, N_pad - pred_len)))
    r, rem = divmod(patch_len, stride)
    if rem == 0:
        hw_r = hw_pad.reshape(num_patch, r, stride, N_pad)
        acc = jnp.pad(hw_r[:, 0], ((0, r - 1), (0, 0), (0, 0)))
        for j in range(1, r):
            acc = acc + jnp.pad(hw_r[:, j], ((j, r - 1 - j), (0, 0), (0, 0)))
        w_eff = jnp.pad(acc.reshape(tgt_len, N_pad), ((s_begin, 0), (0, 0)))
    else:
        t_idx = (s_begin
                 + jnp.arange(num_patch)[:, None] * stride
                 + jnp.arange(patch_len)[None, :]).reshape(-1)
        w_eff = jnp.zeros((S, N_pad), jnp.float32).at[t_idx].add(hw_pad)
    hwT = w_eff.T
    hb_col = jnp.pad(head_b.astype(jnp.float32),
                     (0, N_pad - pred_len)).reshape(N_pad, 1)

    w_l = jnp.pad(affine_weight.astype(jnp.float32), (0, K_pad - K),
                  constant_values=1.0).reshape(1, K_pad)
    b_l = jnp.pad(affine_bias.astype(jnp.float32),
                  (0, K_pad - K)).reshape(1, K_pad)
    invw_l = 1.0 / (w_l + EPS * EPS)

    out = pl.pallas_call(
        _fused_kernel,
        out_shape=jax.ShapeDtypeStruct((B, pred_len, K), out_dtype),
        grid=(B, KB),
        in_specs=[
            pl.BlockSpec((1, S, Kb), lambda i, j: (i, 0, j)),
            pl.BlockSpec((1, Kb), lambda i, j: (0, j)),
            pl.BlockSpec((1, Kb), lambda i, j: (0, j)),
            pl.BlockSpec((1, Kb), lambda i, j: (0, j)),
            pl.BlockSpec((N_pad, S), lambda i, j: (0, 0)),
            pl.BlockSpec((N_pad, 1), lambda i, j: (0, 0)),
        ],
        out_specs=pl.BlockSpec((1, pred_len, Kb), lambda i, j: (i, 0, j)),
        compiler_params=pltpu.CompilerParams(
            dimension_semantics=("parallel", "parallel")),
    )(inputs, w_l, b_l, invw_l, hwT, hb_col)

    xe = jnp.zeros((), jnp.float32)
    xq = jnp.zeros((), jnp.float32)
    return out, xe, xq


def kernel(inputs, affine_weight, affine_bias, head_w, head_b):
    return _rose_forward(inputs, affine_weight, affine_bias, head_w, head_b,
                         patch_len=16, stride=8, pred_len=96)


# speedup vs baseline: 2.7450x; 1.4823x over previous
import functools

import jax
import jax.numpy as jnp
from jax.experimental import pallas as pl
from jax.experimental.pallas import tpu as pltpu

EPS = 1e-5


def _round_up(x, m):
    return ((x + m - 1) // m) * m


def _fused_kernel(x_ref, w_ref, b_ref, invw_ref, hwT_ref, hb_ref, rs_ref,
                  o_ref):
    x = x_ref[0].astype(jnp.float32)
    inv_s = 1.0 / x.shape[0]

    mean = jnp.sum(x, axis=0, keepdims=True) * inv_s
    diff = x - mean
    var = jnp.sum(diff * diff, axis=0, keepdims=True) * inv_s
    inv_std = jax.lax.rsqrt(var + EPS)
    std = (var + EPS) * inv_std

    wx = jnp.dot(hwT_ref[...], x, preferred_element_type=jnp.float32)

    a = w_ref[...] * inv_std
    scale = invw_ref[...] * std
    t = (wx * a + rs_ref[...] * (b_ref[...] - a * mean)
         + (hb_ref[...] - b_ref[...]))
    res = t * scale + mean
    o_ref[0] = res[: o_ref.shape[1], :].astype(o_ref.dtype)


@functools.partial(jax.jit, static_argnames=("patch_len", "stride", "pred_len"))
def _rose_forward(inputs, affine_weight, affine_bias, head_w, head_b,
                  *, patch_len, stride, pred_len):
    B, S, K = inputs.shape
    assert S >= patch_len, "seq_len < patch_len not supported"

    num_patch = (max(S, patch_len) - patch_len) // stride + 1
    tgt_len = patch_len + stride * (num_patch - 1)
    s_begin = S - tgt_len

    N_pad = _round_up(pred_len, 128)
    out_dtype = inputs.dtype

    hw = head_w.astype(jnp.float32)
    hw_pad = jnp.pad(hw, ((0, 0), (0, N_pad - pred_len)))
    r, rem = divmod(patch_len, stride)
    if rem == 0:
        hw_r = hw_pad.reshape(num_patch, r, stride, N_pad)
        acc = jnp.pad(hw_r[:, 0], ((0, r - 1), (0, 0), (0, 0)))
        for j in range(1, r):
            acc = acc + jnp.pad(hw_r[:, j], ((j, r - 1 - j), (0, 0), (0, 0)))
        w_eff = jnp.pad(acc.reshape(tgt_len, N_pad), ((s_begin, 0), (0, 0)))
    else:
        t_idx = (s_begin
                 + jnp.arange(num_patch)[:, None] * stride
                 + jnp.arange(patch_len)[None, :]).reshape(-1)
        w_eff = jnp.zeros((S, N_pad), jnp.float32).at[t_idx].add(hw_pad)
    hwT = w_eff.T
    rs_col = jnp.sum(hw_pad, axis=0).reshape(N_pad, 1)
    hb_col = jnp.pad(head_b.astype(jnp.float32),
                     (0, N_pad - pred_len)).reshape(N_pad, 1)

    w_l = affine_weight.astype(jnp.float32).reshape(1, K)
    b_l = affine_bias.astype(jnp.float32).reshape(1, K)
    invw_l = 1.0 / (w_l + EPS * EPS)

    out = pl.pallas_call(
        _fused_kernel,
        out_shape=jax.ShapeDtypeStruct((B, pred_len, K), out_dtype),
        grid=(B,),
        in_specs=[
            pl.BlockSpec((1, S, K), lambda i: (i, 0, 0)),
            pl.BlockSpec((1, K), lambda i: (0, 0)),
            pl.BlockSpec((1, K), lambda i: (0, 0)),
            pl.BlockSpec((1, K), lambda i: (0, 0)),
            pl.BlockSpec((N_pad, S), lambda i: (0, 0)),
            pl.BlockSpec((N_pad, 1), lambda i: (0, 0)),
            pl.BlockSpec((N_pad, 1), lambda i: (0, 0)),
        ],
        out_specs=pl.BlockSpec((1, pred_len, K), lambda i: (i, 0, 0)),
        compiler_params=pltpu.CompilerParams(
            dimension_semantics=("parallel",)),
    )(inputs, w_l, b_l, invw_l, hwT, hb_col, rs_col)

    xe = jnp.zeros((), jnp.float32)
    xq = jnp.zeros((), jnp.float32)
    return out, xe, xq


def kernel(inputs, affine_weight, affine_bias, head_w, head_b):
    return _rose_forward(inputs, affine_weight, affine_bias, head_w, head_b,
                         patch_len=16, stride=8, pred_len=96)


# speedup vs baseline: 2.7875x; 1.0155x over previous
import functools

import jax
import jax.numpy as jnp
from jax.experimental import pallas as pl
from jax.experimental.pallas import tpu as pltpu

EPS = 1e-5


def _round_up(x, m):
    return ((x + m - 1) // m) * m


def _prep_kernel(hw_ref, hb_ref, hwT_ref, hbrs_ref, *,
                 num_patch, patch_len, stride, s_begin, tgt_len, n_pad):
    hw = hw_ref[...].astype(jnp.float32)
    np_ = hw.shape[1]
    r = patch_len // stride
    hw3 = hw.reshape(num_patch, patch_len, np_)
    acc = None
    for j in range(r):
        sl = hw3[:, j * stride:(j + 1) * stride, :].reshape(
            num_patch * stride, np_)
        term = jnp.pad(sl, ((j * stride, (r - 1 - j) * stride), (0, 0)))
        acc = term if acc is None else acc + term
    w_eff = jnp.pad(acc, ((s_begin, 0), (0, 0)))
    hwT = w_eff.T
    hwT_ref[...] = jnp.pad(hwT, ((0, n_pad - np_), (0, 0)))
    rs = jnp.sum(hwT, axis=1, keepdims=True)
    hb_col = hb_ref[...].astype(jnp.float32).reshape(1, np_).T
    hbrs_ref[...] = jnp.pad(jnp.concatenate([hb_col, rs], axis=1),
                            ((0, n_pad - np_), (0, 0)))


def _main_kernel(x_ref, aw_ref, ab_ref, hwT_ref, hbrs_ref, o_ref):
    x = x_ref[0].astype(jnp.float32)
    inv_s = 1.0 / x.shape[0]
    k = x.shape[1]

    mean = jnp.sum(x, axis=0, keepdims=True) * inv_s
    diff = x - mean
    var = jnp.sum(diff * diff, axis=0, keepdims=True) * inv_s
    inv_std = jax.lax.rsqrt(var + EPS)
    std = (var + EPS) * inv_std

    wx = jnp.dot(hwT_ref[...], x, preferred_element_type=jnp.float32)

    w = aw_ref[...].astype(jnp.float32).reshape(1, k)
    b = ab_ref[...].astype(jnp.float32).reshape(1, k)
    a = w * inv_std
    scale = std / (w + EPS * EPS)
    hb = hbrs_ref[:, 0:1]
    rs = hbrs_ref[:, 1:2]
    t = wx * a + rs * (b - a * mean) + (hb - b)
    res = t * scale + mean
    o_ref[0] = res[: o_ref.shape[1], :].astype(o_ref.dtype)


@functools.partial(jax.jit, static_argnames=("patch_len", "stride", "pred_len"))
def _rose_forward(inputs, affine_weight, affine_bias, head_w, head_b,
                  *, patch_len, stride, pred_len):
    B, S, K = inputs.shape
    assert S >= patch_len, "seq_len < patch_len not supported"

    num_patch = (max(S, patch_len) - patch_len) // stride + 1
    tgt_len = patch_len + stride * (num_patch - 1)
    s_begin = S - tgt_len

    N_pad = _round_up(pred_len, 128)
    out_dtype = inputs.dtype

    if patch_len % stride == 0:
        hwT, hbrs = pl.pallas_call(
            functools.partial(
                _prep_kernel, num_patch=num_patch, patch_len=patch_len,
                stride=stride, s_begin=s_begin, tgt_len=tgt_len, n_pad=N_pad),
            out_shape=(jax.ShapeDtypeStruct((N_pad, S), jnp.float32),
                       jax.ShapeDtypeStruct((N_pad, 2), jnp.float32)),
        )(head_w, head_b)
    else:
        hw_pad = jnp.pad(head_w.astype(jnp.float32),
                         ((0, 0), (0, N_pad - pred_len)))
        t_idx = (s_begin
                 + jnp.arange(num_patch)[:, None] * stride
                 + jnp.arange(patch_len)[None, :]).reshape(-1)
        hwT = jnp.zeros((S, N_pad), jnp.float32).at[t_idx].add(hw_pad).T
        rs = jnp.sum(hw_pad, axis=0).reshape(N_pad, 1)
        hb_col = jnp.pad(head_b.astype(jnp.float32),
                         (0, N_pad - pred_len)).reshape(N_pad, 1)
        hbrs = jnp.concatenate([hb_col, rs], axis=1)

    out = pl.pallas_call(
        _main_kernel,
        out_shape=jax.ShapeDtypeStruct((B, pred_len, K), out_dtype),
        grid=(B,),
        in_specs=[
            pl.BlockSpec((1, S, K), lambda i: (i, 0, 0)),
            pl.BlockSpec((K,), lambda i: (0,)),
            pl.BlockSpec((K,), lambda i: (0,)),
            pl.BlockSpec((N_pad, S), lambda i: (0, 0)),
            pl.BlockSpec((N_pad, 2), lambda i: (0, 0)),
        ],
        out_specs=pl.BlockSpec((1, pred_len, K), lambda i: (i, 0, 0)),
        compiler_params=pltpu.CompilerParams(
            dimension_semantics=("parallel",)),
    )(inputs, affine_weight, affine_bias, hwT, hbrs)

    xe = jnp.zeros((), jnp.float32)
    xq = jnp.zeros((), jnp.float32)
    return out, xe, xq


def kernel(inputs, affine_weight, affine_bias, head_w, head_b):
    return _rose_forward(inputs, affine_weight, affine_bias, head_w, head_b,
                         patch_len=16, stride=8, pred_len=96)
